# Initial kernel scaffold; baseline (speedup 1.0000x reference)
import jax
import jax.numpy as jnp
from jax.experimental import pallas as pl


def kernel(x, edge_index, edge_attr, batch, params):
    raise NotImplementedError("write your pallas kernel here")


# speedup vs baseline: 1.3890x; 1.3890x over previous
import jax
import jax.numpy as jnp
from jax.experimental import pallas as pl

NUM_LAYERS = 3
NUM_GRAPHS = 64


def _ln(h, g, b):
    mu = jnp.mean(h, axis=-1, keepdims=True)
    var = jnp.var(h, axis=-1, keepdims=True)
    return (h - mu) / jnp.sqrt(var + 1e-5) * g + b


def _noise(i, shape):
    k = jax.random.fold_in(jax.random.key(42), i)
    uu = jax.random.uniform(k, shape, minval=1e-6, maxval=1.0 - 1e-6)
    return -jnp.log(-jnp.log(uu))


def kernel(x, edge_index, edge_attr, batch, params):
    n = x.shape[0]
    u, v = edge_index[0], edge_index[1]
    We, be = params['env_attr']
    env_ea = jax.nn.relu(edge_attr @ We + be)
    Wa, ba = params['act_attr']
    act_ea = jax.nn.relu(edge_attr @ Wa + ba)
    Wnode, bnode = params['node']
    h = jax.nn.relu(x @ Wnode + bnode)
    gln, bln = params['ln']

    deg = jax.ops.segment_sum(jnp.ones((u.shape[0],), h.dtype), v, num_segments=n)
    deg = jnp.maximum(deg, 1.0)[:, None]
    env_sum = jax.ops.segment_sum(env_ea, v, num_segments=n)
    act_sum = jax.ops.segment_sum(act_ea, v, num_segments=n)

    ap_in, ap_out = params['in_act'], params['out_act']
    Wr1_i, Wn1_i = ap_in['l1']
    Wr2_i, Wn2_i = ap_in['l2']
    Wr1_o, Wn1_o = ap_out['l1']
    Wr2_o, Wn2_o = ap_out['l2']

    for i in range(NUM_LAYERS):
        hn = _ln(h, gln, bln)
        aggA = jax.ops.segment_sum(hn[u], v, num_segments=n)
        agg1 = (aggA + env_sum) / deg
        h1_in = jax.nn.relu(hn @ Wr1_i + agg1 @ Wn1_i)
        h1_out = jax.nn.relu(hn @ Wr1_o + agg1 @ Wn1_o)
        h1cat = jnp.concatenate([h1_in, h1_out], axis=1)
        aggB = jax.ops.segment_sum(h1cat[u], v, num_segments=n)
        agg2_in = (aggB[:, :16] + act_sum) / deg
        agg2_out = (aggB[:, 16:] + act_sum) / deg
        lin = h1_in @ Wr2_i + agg2_in @ Wn2_i + _noise(2 * i, (n, 2))
        lout = h1_out @ Wr2_o + agg2_out @ Wn2_o + _noise(2 * i + 1, (n, 2))
        a = (lin[:, 0] >= lin[:, 1]).astype(h.dtype)
        b = (lout[:, 0] >= lout[:, 1]).astype(h.dtype)
        hb = hn * b[:, None]
        aggC = (jax.ops.segment_sum(hb[u], v, num_segments=n)
                + jax.ops.segment_sum(b[u][:, None] * env_ea, v, num_segments=n))
        agg = a[:, None] * aggC
        Wr, br, Wnb = params['env_layers'][i]
        h = hn + jax.nn.relu(hn @ Wr + br + agg @ Wnb)

    h = _ln(h, gln, bln)
    W1, b1, W2, b2 = params['dec']
    h = jax.nn.relu(h @ W1 + b1) @ W2 + b2
    graph_emb = h
    Wg1, bg1, Wg2, bg2 = params['gate']
    gate = jax.nn.relu(h @ Wg1 + bg1) @ Wg2 + bg2
    m = jax.ops.segment_max(gate, batch, num_segments=NUM_GRAPHS)
    e = jnp.exp(gate - m[batch])
    denom = jax.ops.segment_sum(e, batch, num_segments=NUM_GRAPHS)
    attn = e / (denom[batch] + 1e-16)
    pooled = jax.ops.segment_sum(attn * h, batch, num_segments=NUM_GRAPHS)
    return (graph_emb, pooled)


# speedup vs baseline: 5.7612x; 4.1477x over previous
import functools

import jax
import jax.numpy as jnp
from jax import lax
from jax.experimental import pallas as pl
from jax.experimental.pallas import tpu as pltpu, tpu_sc as plsc

NUM_LAYERS = 3
NUM_GRAPHS = 64
N = 50000
E = 800000

_info = plsc.get_sparse_core_info()
NC, NS = _info.num_cores, _info.num_subcores
N_PAD = 50048
EDGES_PER_TILE = E // NS
ROWS_PER_TILE = N_PAD // NS
KBLK = 400


def _zero_acc(rows, acc, s, wh, kblk):
    def zfill(i, carry):
        rows[i, pl.ds(0, 16)] = jnp.zeros((16,), jnp.float32)
        if wh > 16:
            rows[i, pl.ds(16, wh - 16)] = jnp.zeros((wh - 16,), jnp.float32)
        return carry
    lax.fori_loop(0, kblk, zfill, jnp.int32(0))
    nfull, rem = ROWS_PER_TILE // kblk, ROWS_PER_TILE % kblk
    def zcopy(j, carry):
        pltpu.sync_copy(rows,
                        acc.at[pl.ds(s * ROWS_PER_TILE + j * kblk, kblk)])
        return carry
    lax.fori_loop(0, nfull, zcopy, jnp.int32(0))
    if rem:
        pltpu.sync_copy(rows.at[pl.ds(0, rem)],
                        acc.at[pl.ds(s * ROWS_PER_TILE + nfull * kblk, rem)])


def _edge_pass(wh, bmul):
    gw = wh + (16 if bmul else 0)
    kblk = 200 if bmul else KBLK
    mesh = plsc.VectorSubcoreMesh(core_axis_name="c", subcore_axis_name="s")

    @functools.partial(
        pl.kernel, mesh=mesh,
        compiler_params=pltpu.CompilerParams(use_tc_tiling_on_sc=False),
        out_type=jax.ShapeDtypeStruct((NC, N_PAD, wh), jnp.float32),
        scratch_types=[
            pltpu.VMEM((kblk,), jnp.int32),
            pltpu.VMEM((kblk,), jnp.int32),
            pltpu.VMEM((kblk, gw), jnp.float32),
            pltpu.VMEM((kblk, wh), jnp.float32),
            pltpu.VMEM_SHARED((N_PAD, wh), jnp.float32),
            pltpu.SemaphoreType.DMA,
        ],
    )
    def k(tables, ue, ve, edata, out, uref, vref, rows, ebuf, acc, sem):
        c = lax.axis_index("c")
        s = lax.axis_index("s")
        _zero_acc(ebuf, acc, s, wh, kblk)
        plsc.subcore_barrier()

        nv = wh // 16

        def body(j, carry):
            base = s * EDGES_PER_TILE + j * kblk
            pltpu.sync_copy(ue.at[pl.ds(base, kblk)], uref)
            pltpu.sync_copy(ve.at[pl.ds(base, kblk)], vref)
            pltpu.sync_copy(edata.at[c, pl.ds(base, kblk)], ebuf)
            pltpu.async_copy(tables.at[c].at[uref], rows, sem).wait()

            def upd(i, carry):
                if bmul:
                    bb = rows[i, pl.ds(wh, 16)]
                    for d in range(nv):
                        ebuf[i, pl.ds(16 * d, 16)] = (
                            rows[i, pl.ds(16 * d, 16)]
                            + ebuf[i, pl.ds(16 * d, 16)]) * bb
                else:
                    for d in range(nv):
                        ebuf[i, pl.ds(16 * d, 16)] = (
                            rows[i, pl.ds(16 * d, 16)]
                            + ebuf[i, pl.ds(16 * d, 16)])
                return carry
            lax.fori_loop(0, kblk, upd, jnp.int32(0))
            pltpu.sync_copy(ebuf, acc.at[vref], add=True)
            return carry
        lax.fori_loop(0, EDGES_PER_TILE // kblk, body, jnp.int32(0))
        plsc.subcore_barrier()
        pltpu.sync_copy(
            acc.at[pl.ds(s * ROWS_PER_TILE, ROWS_PER_TILE)],
            out.at[c, pl.ds(s * ROWS_PER_TILE, ROWS_PER_TILE)])

    return k


_edge_pass_32 = _edge_pass(32, False)
_edge_pass_16 = _edge_pass(16, False)
_edge_pass_32b = _edge_pass(32, True)

DEG_KBLK = 1000
DEG_EPT = E // NC // NS

_deg_mesh = plsc.VectorSubcoreMesh(core_axis_name="c", subcore_axis_name="s")


@functools.partial(
    pl.kernel, mesh=_deg_mesh,
    compiler_params=pltpu.CompilerParams(use_tc_tiling_on_sc=False),
    out_type=jax.ShapeDtypeStruct((NC, N_PAD), jnp.float32),
    scratch_types=[
        pltpu.VMEM((DEG_KBLK,), jnp.int32),
        pltpu.VMEM((DEG_KBLK,), jnp.float32),
        pltpu.VMEM_SHARED((N_PAD,), jnp.float32),
        pltpu.SemaphoreType.DMA,
    ],
)
def _deg_pass(ve, out, vref, ones, acc, sem):
    c = lax.axis_index("c")
    s = lax.axis_index("s")
    def zfill(i, carry):
        ones[pl.ds(16 * i, 16)] = jnp.zeros((16,), jnp.float32)
        return carry
    lax.fori_loop(0, DEG_KBLK // 16, zfill, jnp.int32(0))
    ones[pl.ds(DEG_KBLK - 16, 16)] = jnp.zeros((16,), jnp.float32)
    nfull = ROWS_PER_TILE // DEG_KBLK
    def zcopy(j, carry):
        pltpu.sync_copy(ones, acc.at[pl.ds(s * ROWS_PER_TILE + j * DEG_KBLK,
                                           DEG_KBLK)])
        return carry
    lax.fori_loop(0, nfull, zcopy, jnp.int32(0))
    rem = ROWS_PER_TILE % DEG_KBLK
    if rem:
        pltpu.sync_copy(ones.at[pl.ds(0, rem)],
                        acc.at[pl.ds(s * ROWS_PER_TILE + nfull * DEG_KBLK, rem)])
    def ofill(i, carry):
        ones[pl.ds(16 * i, 16)] = jnp.ones((16,), jnp.float32)
        return carry
    lax.fori_loop(0, DEG_KBLK // 16, ofill, jnp.int32(0))
    ones[pl.ds(DEG_KBLK - 16, 16)] = jnp.ones((16,), jnp.float32)
    plsc.subcore_barrier()

    def body(j, carry):
        base = (c * NS + s) * DEG_EPT + j * DEG_KBLK
        pltpu.sync_copy(ve.at[pl.ds(base, DEG_KBLK)], vref)
        pltpu.sync_copy(ones, acc.at[vref], add=True)
        return carry
    lax.fori_loop(0, DEG_EPT // DEG_KBLK, body, jnp.int32(0))
    plsc.subcore_barrier()
    pltpu.sync_copy(acc.at[pl.ds(s * ROWS_PER_TILE, ROWS_PER_TILE)],
                    out.at[c, pl.ds(s * ROWS_PER_TILE, ROWS_PER_TILE)])


def _ln(h, g, b):
    mu = jnp.mean(h, axis=-1, keepdims=True)
    var = jnp.var(h, axis=-1, keepdims=True)
    return (h - mu) / jnp.sqrt(var + 1e-5) * g + b


def _noise(i, shape):
    k = jax.random.fold_in(jax.random.key(42), i)
    uu = jax.random.uniform(k, shape, minval=1e-6, maxval=1.0 - 1e-6)
    return -jnp.log(-jnp.log(uu))


def kernel(x, edge_index, edge_attr, batch, params):
    n = x.shape[0]
    u, v = edge_index[0], edge_index[1]
    We, be = params['env_attr']
    env_ea = jax.nn.relu(edge_attr @ We + be)
    Wa, ba = params['act_attr']
    act_ea = jax.nn.relu(edge_attr @ Wa + ba)
    Wnode, bnode = params['node']
    h = jax.nn.relu(x @ Wnode + bnode)
    gln, bln = params['ln']

    env_halves = jnp.stack([env_ea[:, :32], env_ea[:, 32:]])
    act2 = jnp.stack([act_ea, act_ea])

    degp = _deg_pass(v)
    inv_deg = (1.0 / jnp.maximum(degp[0, :N] + degp[1, :N], 1.0))[:, None]

    ap_in, ap_out = params['in_act'], params['out_act']
    Wr1_i, Wn1_i = ap_in['l1']
    Wr2_i, Wn2_i = ap_in['l2']
    Wr1_o, Wn1_o = ap_out['l1']
    Wr2_o, Wn2_o = ap_out['l2']

    for i in range(NUM_LAYERS):
        hn = _ln(h, gln, bln)
        hn_halves = jnp.stack([hn[:, :32], hn[:, 32:]])
        aggA2 = _edge_pass_32(hn_halves, u, v, env_halves)
        agg1 = jnp.concatenate([aggA2[0, :N], aggA2[1, :N]], axis=1) * inv_deg
        h1_in = jax.nn.relu(hn @ Wr1_i + agg1 @ Wn1_i)
        h1_out = jax.nn.relu(hn @ Wr1_o + agg1 @ Wn1_o)
        h1_tab = jnp.stack([h1_in, h1_out])
        aggB2 = _edge_pass_16(h1_tab, u, v, act2)
        agg2_in = aggB2[0, :N] * inv_deg
        agg2_out = aggB2[1, :N] * inv_deg
        lin = h1_in @ Wr2_i + agg2_in @ Wn2_i + _noise(2 * i, (n, 2))
        lout = h1_out @ Wr2_o + agg2_out @ Wn2_o + _noise(2 * i + 1, (n, 2))
        a = (lin[:, 0] >= lin[:, 1]).astype(h.dtype)
        b = (lout[:, 0] >= lout[:, 1]).astype(h.dtype)
        bpad = jnp.tile(b[:, None], (1, 16))
        hnb = jnp.stack(
            [jnp.concatenate([hn[:, :32], bpad], axis=1),
             jnp.concatenate([hn[:, 32:], bpad], axis=1)])
        aggC2 = _edge_pass_32b(hnb, u, v, env_halves)
        aggC = jnp.concatenate([aggC2[0, :N], aggC2[1, :N]], axis=1)
        agg = a[:, None] * aggC
        Wr, br, Wnb = params['env_layers'][i]
        h = hn + jax.nn.relu(hn @ Wr + br + agg @ Wnb)

    h = _ln(h, gln, bln)
    W1, b1, W2, b2 = params['dec']
    h = jax.nn.relu(h @ W1 + b1) @ W2 + b2
    graph_emb = h
    Wg1, bg1, Wg2, bg2 = params['gate']
    gate = jax.nn.relu(h @ Wg1 + bg1) @ Wg2 + bg2
    m = jax.ops.segment_max(gate, batch, num_segments=NUM_GRAPHS)
    e = jnp.exp(gate - m[batch])
    denom = jax.ops.segment_sum(e, batch, num_segments=NUM_GRAPHS)
    attn = e / (denom[batch] + 1e-16)
    pooled = jax.ops.segment_sum(attn * h, batch, num_segments=NUM_GRAPHS)
    return (graph_emb, pooled)
